# baseline (speedup 1.0000x reference)
import functools

import jax
import jax.numpy as jnp
from jax import lax
from jax.experimental import pallas as pl
from jax.experimental.pallas import tpu as pltpu
from jax.experimental.pallas import tpu_sc as plsc

f32 = jnp.float32
i32 = jnp.int32

N_NODES = 10000
HIDDEN = 768
N_EDGES = 100000
N_PAIRS = 8192
W = 16

NC = 2
NS = 16
NW = NC * NS
NPAD = 10240
ROWS_PER_SUB = NPAD // NS
EPAD = 102400
ECH = EPAD // (NW * 128)
PCH = N_PAIRS // (NW * 128)

BLK = 1000


def _prep_body(w1_ref, wl_ref, wr_ref, wc_ref, bs_ref, bc_ref,
               w1b_ref, c_ref, k_ref):
    w1b_ref[...] = w1_ref[...].astype(jnp.bfloat16)
    wc6 = jnp.concatenate([wc_ref[0:HIDDEN, :], wc_ref[HIDDEN:, :]], axis=1)
    z = jnp.zeros((HIDDEN, 2), f32)
    wcm = jnp.concatenate([wc6, z, z, z, z, z], axis=1)
    wcr = jnp.concatenate([z, z, z, z, wc6, z], axis=1)
    c_ref[...] = (jnp.dot(wl_ref[...], wcm, preferred_element_type=f32)
                  + jnp.dot(wr_ref[...], wcr, preferred_element_type=f32)
                  + wcr).astype(jnp.bfloat16)
    kv = jnp.dot(bs_ref[...], wcm, preferred_element_type=f32)
    r16 = lax.broadcasted_iota(i32, (16, 16), 0)
    c16 = lax.broadcasted_iota(i32, (16, 16), 1)
    shift3 = jnp.where(r16 == c16 + 3, 1.0, 0.0).astype(f32)
    ksh = jnp.dot(kv, shift3, preferred_element_type=f32)
    bcp = jnp.concatenate([bc_ref[...], jnp.zeros((1, 13), f32)], axis=1)
    col = lax.broadcasted_iota(i32, (1, 16), 1)
    k_ref[...] = jnp.where(col < 3, kv + ksh + bcp, 0.0)


def _prep(W1, Wl, Wr, Wc, bsr, bcp):
    return pl.pallas_call(
        _prep_body,
        out_shape=[jax.ShapeDtypeStruct((HIDDEN, HIDDEN), jnp.bfloat16),
                   jax.ShapeDtypeStruct((HIDDEN, W), jnp.bfloat16),
                   jax.ShapeDtypeStruct((1, 16), f32)],
    )(W1, Wl, Wr, Wc, bsr, bcp)


def _dense_body(ne_ref, w1b_ref, b1_ref, c_ref, x_ref, xm_ref):
    lhs = ne_ref[...].astype(jnp.bfloat16)
    h = jnp.dot(lhs, w1b_ref[...], preferred_element_type=f32)
    h = jnp.maximum(h + b1_ref[...], 0.0)
    x = jnp.dot(h.astype(jnp.bfloat16), c_ref[...],
                preferred_element_type=f32)
    col = lax.broadcasted_iota(i32, x.shape, 1)
    x = jnp.where(col == 6, 1.0, x)
    x_ref[...] = x
    xm_ref[...] = x[:, 0:8]


def _dense(ne, W1bf, b1r, C):
    nblk = N_NODES // BLK
    zero = lambda i: (0, 0)
    return pl.pallas_call(
        _dense_body,
        grid=(nblk,),
        in_specs=[
            pl.BlockSpec((BLK, HIDDEN), lambda i: (i, 0)),
            pl.BlockSpec((HIDDEN, HIDDEN), zero),
            pl.BlockSpec((1, HIDDEN), zero),
            pl.BlockSpec((HIDDEN, W), zero),
        ],
        out_specs=[pl.BlockSpec((BLK, W), lambda i: (i, 0)),
                   pl.BlockSpec((BLK, 8), lambda i: (i, 0))],
        out_shape=[jax.ShapeDtypeStruct((N_NODES, W), f32),
                   jax.ShapeDtypeStruct((N_NODES, 8), f32)],
        compiler_params=pltpu.CompilerParams(
            dimension_semantics=("parallel",)),
    )(ne, W1bf, b1r, C)


_MESH = plsc.VectorSubcoreMesh(core_axis_name="c", subcore_axis_name="s")
_SC_PARAMS = pltpu.CompilerParams(use_tc_tiling_on_sc=False)


@functools.partial(
    pl.kernel,
    out_type=jax.ShapeDtypeStruct((NPAD, W), f32),
    mesh=_MESH,
    scratch_types=[
        pltpu.VMEM((ECH, 128), i32),
        pltpu.VMEM((ECH, 128), i32),
        pltpu.VMEM((ECH, 128, 8), f32),
        pltpu.VMEM_SHARED((NPAD, 8), f32),
        pltpu.SemaphoreType.DMA,
        pltpu.SemaphoreType.DMA,
    ],
    compiler_params=_SC_PARAMS,
)
def _s1(src_hbm, dst_hbm, x_hbm, zeros_hbm, p_hbm,
        src_v, dst_v, rows_v, seg_sh, sem, sem2):
    cid = lax.axis_index("c")
    sid = lax.axis_index("s")
    wid = sid * NC + cid

    zrows = pl.ds(sid * ROWS_PER_SUB, ROWS_PER_SUB)
    pltpu.sync_copy(zeros_hbm.at[zrows], seg_sh.at[zrows])

    pltpu.sync_copy(src_hbm.at[wid], src_v)
    pltpu.sync_copy(dst_hbm.at[wid], dst_v)
    plsc.subcore_barrier()

    gds = [pltpu.async_copy(x_hbm.at[src_v.at[j]], rows_v.at[j], sem)
           for j in range(ECH)]
    sds = []
    for j in range(ECH):
        gds[j].wait()
        sds.append(pltpu.async_copy(rows_v.at[j], seg_sh.at[dst_v.at[j]],
                                    sem2, add=True))
    for d in sds:
        d.wait()
    plsc.subcore_barrier()

    rows = pl.ds(sid * ROWS_PER_SUB, ROWS_PER_SUB)

    @pl.when(cid == 0)
    def _():
        pltpu.sync_copy(seg_sh.at[rows], p_hbm.at[rows, pl.ds(0, 8)])

    @pl.when(cid == 1)
    def _():
        pltpu.sync_copy(seg_sh.at[rows], p_hbm.at[rows, pl.ds(8, 8)])


_TAKE_DN = lax.GatherDimensionNumbers(
    offset_dims=(), collapsed_slice_dims=(0,), start_index_map=(0,))


def _lane_take(x, idxv):
    return lax.gather(x, idxv[:, None], _TAKE_DN, slice_sizes=(1,),
                      mode=lax.GatherScatterMode.PROMISE_IN_BOUNDS)


@functools.partial(
    pl.kernel,
    out_type=jax.ShapeDtypeStruct((N_PAIRS, W), f32),
    mesh=_MESH,
    scratch_types=[
        pltpu.VMEM((PCH, 128), i32),
        pltpu.VMEM((PCH, 128), i32),
        pltpu.VMEM((128, W), f32),
        pltpu.VMEM((128, W), f32),
        pltpu.VMEM((128, W), f32),
        pltpu.VMEM((128, W), f32),
        pltpu.VMEM((128, W), f32),
        pltpu.VMEM((16,), f32),
        pltpu.SemaphoreType.DMA,
    ],
    compiler_params=_SC_PARAMS,
)
def _s2(i0_hbm, i1_hbm, x_hbm, p_hbm, k_hbm, out_hbm,
        i0_v, i1_v, x0_v, a0_v, x1_v, a1_v, out_v, kv, sem):
    cid = lax.axis_index("c")
    sid = lax.axis_index("s")
    wid = sid * NC + cid

    pltpu.sync_copy(k_hbm, kv)
    pltpu.sync_copy(i0_hbm.at[wid], i0_v)
    pltpu.sync_copy(i1_hbm.at[wid], i1_v)

    lanes = lax.iota(i32, 16)
    six = jnp.full((16,), 6, i32)
    sh8 = jnp.minimum(lanes + 8, 15)
    sh3 = jnp.minimum(lanes + 3, 15)

    for j in range(PCH):
        d0 = pltpu.async_copy(x_hbm.at[i0_v.at[j]], x0_v, sem)
        d1 = pltpu.async_copy(p_hbm.at[i0_v.at[j]], a0_v, sem)
        d2 = pltpu.async_copy(x_hbm.at[i1_v.at[j]], x1_v, sem)
        d3 = pltpu.async_copy(p_hbm.at[i1_v.at[j]], a1_v, sem)
        d0.wait(); d1.wait(); d2.wait(); d3.wait()

        kcst = kv[...]

        def _pair(p, carry):
            v0 = a0_v[p, :]
            s0 = v0 + _lane_take(v0, sh8)
            rec0 = 1.0 / jnp.maximum(_lane_take(s0, six), 1.0)
            tl = s0 * rec0 + _lane_take(x0_v[p, :], sh8)
            v1 = a1_v[p, :]
            s1 = v1 + _lane_take(v1, sh8)
            rec1 = 1.0 / jnp.maximum(_lane_take(s1, six), 1.0)
            tr = s1 * rec1 + _lane_take(x1_v[p, :], sh8)
            out_v[p, :] = tl + _lane_take(tr, sh3) + kcst
            return carry
        lax.fori_loop(0, 128, _pair, 0)

        base = (wid * PCH + j) * 128
        pltpu.sync_copy(out_v, out_hbm.at[pl.ds(base, 128)])


def kernel(node_embeddings, W1, b1, Wl, Wr, b_sage, Wc, bc, idx, edge_index):
    ne = node_embeddings.astype(f32)
    W1bf, C, K = _prep(W1, Wl, Wr, Wc, b_sage.reshape(1, HIDDEN),
                       bc.reshape(1, 3))
    X, Xm = _dense(ne, W1bf, b1.reshape(1, HIDDEN), C)

    src = edge_index[0].astype(i32)
    dst = edge_index[1].astype(i32)
    pad = EPAD - N_EDGES
    srcp = jnp.concatenate([src, jnp.zeros((pad,), i32)]).reshape(NW, ECH, 128)
    dstp = jnp.concatenate([dst, jnp.full((pad,), NPAD - 1, i32)]
                           ).reshape(NW, ECH, 128)
    zeros8 = jnp.zeros((NPAD, 8), f32)
    p = _s1(srcp, dstp, Xm, zeros8)

    i0 = idx[:, 0].astype(i32).reshape(NW, PCH, 128)
    i1 = idx[:, 1].astype(i32).reshape(NW, PCH, 128)
    out = _s2(i0, i1, X, p, K.reshape(16))
    return out[:, :3]


# speedup vs baseline: 1.0749x; 1.0749x over previous
import functools

import jax
import jax.numpy as jnp
from jax import lax
from jax.experimental import pallas as pl
from jax.experimental.pallas import tpu as pltpu
from jax.experimental.pallas import tpu_sc as plsc

f32 = jnp.float32
i32 = jnp.int32

N_NODES = 10000
HIDDEN = 768
N_EDGES = 100000
N_PAIRS = 8192
W = 16

NC = 2
NS = 16
NW = NC * NS
NPAD = 10240
ROWS_PER_SUB = NPAD // NS
EPAD = 102400
ECH = EPAD // (NW * 128)
PCH = N_PAIRS // (NW * 128)

BLK = 1000


def _dense_body(ne_ref, w1_ref, b1_ref, wl_ref, wr_ref, wc_ref, bs_ref,
                bc_ref, x_ref, xm_ref, k_ref, c_vmem, w1b_vmem):
    i = pl.program_id(0)

    @pl.when(i == 0)
    def _():
        w1b_vmem[...] = w1_ref[...].astype(jnp.bfloat16)
        wc6 = jnp.concatenate([wc_ref[0:HIDDEN, :], wc_ref[HIDDEN:, :]], axis=1)
        z = jnp.zeros((HIDDEN, 2), f32)
        wcm = jnp.concatenate([wc6, z, z, z, z, z], axis=1)
        wcr = jnp.concatenate([z, z, z, z, wc6, z], axis=1)
        c_vmem[...] = (jnp.dot(wl_ref[...], wcm, preferred_element_type=f32)
                       + jnp.dot(wr_ref[...], wcr, preferred_element_type=f32)
                       + wcr)
        kv = jnp.dot(bs_ref[...], wcm, preferred_element_type=f32)
        r16 = lax.broadcasted_iota(i32, (16, 16), 0)
        c16 = lax.broadcasted_iota(i32, (16, 16), 1)
        shift3 = jnp.where(r16 == c16 + 3, 1.0, 0.0).astype(f32)
        ksh = jnp.dot(kv, shift3, preferred_element_type=f32)
        bcp = jnp.concatenate([bc_ref[...], jnp.zeros((1, 13), f32)], axis=1)
        col = lax.broadcasted_iota(i32, (1, 16), 1)
        k_ref[...] = jnp.where(col < 3, kv + ksh + bcp, 0.0)

    lhs = ne_ref[...].astype(jnp.bfloat16)
    h = jnp.dot(lhs, w1b_vmem[...], preferred_element_type=f32)
    h = jnp.maximum(h + b1_ref[...], 0.0)
    x = jnp.dot(h.astype(jnp.bfloat16), c_vmem[...].astype(jnp.bfloat16),
                preferred_element_type=f32)
    col = lax.broadcasted_iota(i32, x.shape, 1)
    x = jnp.where(col == 6, 1.0, x)
    x_ref[...] = x
    xm_ref[...] = x[:, 0:8]


def _dense(ne, W1b, b1r, Wl, Wr, Wc, bsr, bcp):
    nblk = N_NODES // BLK
    zero = lambda i: (0, 0)
    return pl.pallas_call(
        _dense_body,
        grid=(nblk,),
        in_specs=[
            pl.BlockSpec((BLK, HIDDEN), lambda i: (i, 0)),
            pl.BlockSpec((HIDDEN, HIDDEN), zero),
            pl.BlockSpec((1, HIDDEN), zero),
            pl.BlockSpec((HIDDEN, HIDDEN), zero),
            pl.BlockSpec((HIDDEN, HIDDEN), zero),
            pl.BlockSpec((2 * HIDDEN, 3), zero),
            pl.BlockSpec((1, HIDDEN), zero),
            pl.BlockSpec((1, 3), zero),
        ],
        out_specs=[pl.BlockSpec((BLK, W), lambda i: (i, 0)),
                   pl.BlockSpec((BLK, 8), lambda i: (i, 0)),
                   pl.BlockSpec((1, 16), zero)],
        out_shape=[jax.ShapeDtypeStruct((N_NODES, W), f32),
                   jax.ShapeDtypeStruct((N_NODES, 8), f32),
                   jax.ShapeDtypeStruct((1, 16), f32)],
        scratch_shapes=[pltpu.VMEM((HIDDEN, W), f32),
                        pltpu.VMEM((HIDDEN, HIDDEN), jnp.bfloat16)],
    )(ne, W1b, b1r, Wl, Wr, Wc, bsr, bcp)


_MESH = plsc.VectorSubcoreMesh(core_axis_name="c", subcore_axis_name="s")
_SC_PARAMS = pltpu.CompilerParams(use_tc_tiling_on_sc=False)


@functools.partial(
    pl.kernel,
    out_type=jax.ShapeDtypeStruct((NPAD, W), f32),
    mesh=_MESH,
    scratch_types=[
        pltpu.VMEM((ECH, 128), i32),
        pltpu.VMEM((ECH, 128), i32),
        pltpu.VMEM((ECH, 128, 8), f32),
        pltpu.VMEM_SHARED((NPAD, 8), f32),
        pltpu.SemaphoreType.DMA,
        pltpu.SemaphoreType.DMA,
    ],
    compiler_params=_SC_PARAMS,
)
def _s1(ei_hbm, x_hbm, zeros_hbm, p_hbm,
        src_v, dst_v, rows_v, seg_sh, sem, sem2):
    cid = lax.axis_index("c")
    sid = lax.axis_index("s")
    wid = sid * NC + cid

    zrows = pl.ds(sid * ROWS_PER_SUB, ROWS_PER_SUB)
    pltpu.sync_copy(zeros_hbm.at[zrows], seg_sh.at[zrows])

    pltpu.sync_copy(ei_hbm.at[0, wid], src_v)
    pltpu.sync_copy(ei_hbm.at[1, wid], dst_v)
    plsc.subcore_barrier()

    gds = [pltpu.async_copy(x_hbm.at[src_v.at[j]], rows_v.at[j], sem)
           for j in range(ECH)]
    sds = []
    for j in range(ECH):
        gds[j].wait()
        sds.append(pltpu.async_copy(rows_v.at[j], seg_sh.at[dst_v.at[j]],
                                    sem2, add=True))
    for d in sds:
        d.wait()
    plsc.subcore_barrier()

    rows = pl.ds(sid * ROWS_PER_SUB, ROWS_PER_SUB)

    @pl.when(cid == 0)
    def _():
        pltpu.sync_copy(seg_sh.at[rows], p_hbm.at[rows, pl.ds(0, 8)])

    @pl.when(cid == 1)
    def _():
        pltpu.sync_copy(seg_sh.at[rows], p_hbm.at[rows, pl.ds(8, 8)])


_TAKE_DN = lax.GatherDimensionNumbers(
    offset_dims=(), collapsed_slice_dims=(0,), start_index_map=(0,))


def _lane_take(x, idxv):
    return lax.gather(x, idxv[:, None], _TAKE_DN, slice_sizes=(1,),
                      mode=lax.GatherScatterMode.PROMISE_IN_BOUNDS)


@functools.partial(
    pl.kernel,
    out_type=jax.ShapeDtypeStruct((N_PAIRS, W), f32),
    mesh=_MESH,
    scratch_types=[
        pltpu.VMEM((PCH, 128), i32),
        pltpu.VMEM((PCH, 128), i32),
        pltpu.VMEM((128, W), f32),
        pltpu.VMEM((128, W), f32),
        pltpu.VMEM((128, W), f32),
        pltpu.VMEM((128, W), f32),
        pltpu.VMEM((128, W), f32),
        pltpu.VMEM((16,), f32),
        pltpu.SemaphoreType.DMA,
    ],
    compiler_params=_SC_PARAMS,
)
def _s2(idx_hbm, x_hbm, p_hbm, k_hbm, out_hbm,
        i0_v, i1_v, x0_v, a0_v, x1_v, a1_v, out_v, kv, sem):
    cid = lax.axis_index("c")
    sid = lax.axis_index("s")
    wid = sid * NC + cid

    pltpu.sync_copy(k_hbm, kv)
    pltpu.sync_copy(idx_hbm.at[0, wid], i0_v)
    pltpu.sync_copy(idx_hbm.at[1, wid], i1_v)

    lanes = lax.iota(i32, 16)
    six = jnp.full((16,), 6, i32)
    sh8 = jnp.minimum(lanes + 8, 15)
    sh3 = jnp.minimum(lanes + 3, 15)

    for j in range(PCH):
        d0 = pltpu.async_copy(x_hbm.at[i0_v.at[j]], x0_v, sem)
        d1 = pltpu.async_copy(p_hbm.at[i0_v.at[j]], a0_v, sem)
        d2 = pltpu.async_copy(x_hbm.at[i1_v.at[j]], x1_v, sem)
        d3 = pltpu.async_copy(p_hbm.at[i1_v.at[j]], a1_v, sem)
        d0.wait(); d1.wait(); d2.wait(); d3.wait()

        kcst = kv[...]

        def _pair(p, carry):
            v0 = a0_v[p, :]
            s0 = v0 + _lane_take(v0, sh8)
            rec0 = 1.0 / jnp.maximum(_lane_take(s0, six), 1.0)
            tl = s0 * rec0 + _lane_take(x0_v[p, :], sh8)
            v1 = a1_v[p, :]
            s1 = v1 + _lane_take(v1, sh8)
            rec1 = 1.0 / jnp.maximum(_lane_take(s1, six), 1.0)
            tr = s1 * rec1 + _lane_take(x1_v[p, :], sh8)
            out_v[p, :] = tl + _lane_take(tr, sh3) + kcst
            return carry
        lax.fori_loop(0, 128, _pair, 0)

        base = (wid * PCH + j) * 128
        pltpu.sync_copy(out_v, out_hbm.at[pl.ds(base, 128)])


def kernel(node_embeddings, W1, b1, Wl, Wr, b_sage, Wc, bc, idx, edge_index):
    ne = node_embeddings.astype(f32)
    X, Xm, K = _dense(ne, W1, b1.reshape(1, HIDDEN),
                      Wl, Wr, Wc, b_sage.reshape(1, HIDDEN), bc.reshape(1, 3))

    pad = EPAD - N_EDGES
    padv = jnp.broadcast_to(jnp.array([[0], [NPAD - 1]], i32), (2, pad))
    ei = jnp.concatenate([edge_index.astype(i32), padv], axis=1
                         ).reshape(2, NW, ECH, 128)
    zeros8 = jnp.zeros((NPAD, 8), f32)
    p = _s1(ei, Xm, zeros8)

    idxT = idx.astype(i32).T.reshape(2, NW, PCH, 128)
    out = _s2(idxT, X, p, K.reshape(16))
    return out[:, :3]


# speedup vs baseline: 1.1020x; 1.0252x over previous
import functools

import jax
import jax.numpy as jnp
from jax import lax
from jax.experimental import pallas as pl
from jax.experimental.pallas import tpu as pltpu
from jax.experimental.pallas import tpu_sc as plsc

f32 = jnp.float32
i32 = jnp.int32

N_NODES = 10000
HIDDEN = 768
N_EDGES = 100000
N_PAIRS = 8192
W = 16

NC = 2
NS = 16
NW = NC * NS
NPAD = 10240
ROWS_PER_SUB = NPAD // NS
EPAD = 102400
ECH = EPAD // (NW * 128)
PCH = N_PAIRS // (NW * 128)

BLK = 2000


def _dense_body(ne_ref, w1_ref, b1_ref, wl_ref, wr_ref, wc_ref, bs_ref,
                bc_ref, x_ref, xm_ref, k_ref, c_vmem, w1b_vmem):
    i = pl.program_id(0)

    @pl.when(i == 0)
    def _():
        w1b_vmem[...] = w1_ref[...].astype(jnp.bfloat16)
        wc6 = jnp.concatenate([wc_ref[0:HIDDEN, :], wc_ref[HIDDEN:, :]], axis=1)
        z = jnp.zeros((HIDDEN, 2), f32)
        wcm = jnp.concatenate([wc6, z, z, z, z, z], axis=1)
        wcr = jnp.concatenate([z, z, z, z, wc6, z], axis=1)
        c_vmem[...] = (jnp.dot(wl_ref[...], wcm, preferred_element_type=f32)
                       + jnp.dot(wr_ref[...], wcr, preferred_element_type=f32)
                       + wcr)
        kv = jnp.dot(bs_ref[...], wcm, preferred_element_type=f32)
        r16 = lax.broadcasted_iota(i32, (16, 16), 0)
        c16 = lax.broadcasted_iota(i32, (16, 16), 1)
        shift3 = jnp.where(r16 == c16 + 3, 1.0, 0.0).astype(f32)
        ksh = jnp.dot(kv, shift3, preferred_element_type=f32)
        bcp = jnp.concatenate([bc_ref[...], jnp.zeros((1, 13), f32)], axis=1)
        col = lax.broadcasted_iota(i32, (1, 16), 1)
        k_ref[...] = jnp.where(col < 3, kv + ksh + bcp, 0.0)

    lhs = ne_ref[...].astype(jnp.bfloat16)
    h = jnp.dot(lhs, w1b_vmem[...], preferred_element_type=f32)
    h = jnp.maximum(h + b1_ref[...], 0.0)
    x = jnp.dot(h.astype(jnp.bfloat16), c_vmem[...].astype(jnp.bfloat16),
                preferred_element_type=f32)
    col = lax.broadcasted_iota(i32, x.shape, 1)
    x = jnp.where(col == 6, 1.0, x)
    x_ref[...] = x
    xm_ref[...] = x[:, 0:8]


def _dense(ne, W1b, b1r, Wl, Wr, Wc, bsr, bcp):
    nblk = N_NODES // BLK
    zero = lambda i: (0, 0)
    return pl.pallas_call(
        _dense_body,
        grid=(nblk,),
        in_specs=[
            pl.BlockSpec((BLK, HIDDEN), lambda i: (i, 0)),
            pl.BlockSpec((HIDDEN, HIDDEN), zero),
            pl.BlockSpec((1, HIDDEN), zero),
            pl.BlockSpec((HIDDEN, HIDDEN), zero),
            pl.BlockSpec((HIDDEN, HIDDEN), zero),
            pl.BlockSpec((2 * HIDDEN, 3), zero),
            pl.BlockSpec((1, HIDDEN), zero),
            pl.BlockSpec((1, 3), zero),
        ],
        out_specs=[pl.BlockSpec((BLK, W), lambda i: (i, 0)),
                   pl.BlockSpec((BLK, 8), lambda i: (i, 0)),
                   pl.BlockSpec((1, 16), zero)],
        out_shape=[jax.ShapeDtypeStruct((N_NODES, W), f32),
                   jax.ShapeDtypeStruct((N_NODES, 8), f32),
                   jax.ShapeDtypeStruct((1, 16), f32)],
        scratch_shapes=[pltpu.VMEM((HIDDEN, W), f32),
                        pltpu.VMEM((HIDDEN, HIDDEN), jnp.bfloat16)],
    )(ne, W1b, b1r, Wl, Wr, Wc, bsr, bcp)


_MESH = plsc.VectorSubcoreMesh(core_axis_name="c", subcore_axis_name="s")
_SC_PARAMS = pltpu.CompilerParams(use_tc_tiling_on_sc=False)


@functools.partial(
    pl.kernel,
    out_type=jax.ShapeDtypeStruct((NPAD, W), f32),
    mesh=_MESH,
    scratch_types=[
        pltpu.VMEM((ECH, 128), i32),
        pltpu.VMEM((ECH, 128), i32),
        pltpu.VMEM((ECH, 128, 8), f32),
        pltpu.VMEM_SHARED((NPAD, 8), f32),
        pltpu.SemaphoreType.DMA,
        pltpu.SemaphoreType.DMA,
    ],
    compiler_params=_SC_PARAMS,
)
def _s1(ei_hbm, x_hbm, zeros_hbm, p_hbm,
        src_v, dst_v, rows_v, seg_sh, sem, sem2):
    cid = lax.axis_index("c")
    sid = lax.axis_index("s")
    wid = sid * NC + cid

    zrows = pl.ds(sid * ROWS_PER_SUB, ROWS_PER_SUB)
    pltpu.sync_copy(zeros_hbm.at[zrows], seg_sh.at[zrows])

    pltpu.sync_copy(ei_hbm.at[0, wid], src_v)
    pltpu.sync_copy(ei_hbm.at[1, wid], dst_v)
    plsc.subcore_barrier()

    gds = [pltpu.async_copy(x_hbm.at[src_v.at[j]], rows_v.at[j], sem)
           for j in range(ECH)]
    sds = []
    for j in range(ECH):
        gds[j].wait()
        sds.append(pltpu.async_copy(rows_v.at[j], seg_sh.at[dst_v.at[j]],
                                    sem2, add=True))
    for d in sds:
        d.wait()
    plsc.subcore_barrier()

    rows = pl.ds(sid * ROWS_PER_SUB, ROWS_PER_SUB)

    @pl.when(cid == 0)
    def _():
        pltpu.sync_copy(seg_sh.at[rows], p_hbm.at[rows, pl.ds(0, 8)])

    @pl.when(cid == 1)
    def _():
        pltpu.sync_copy(seg_sh.at[rows], p_hbm.at[rows, pl.ds(8, 8)])


_TAKE_DN = lax.GatherDimensionNumbers(
    offset_dims=(), collapsed_slice_dims=(0,), start_index_map=(0,))


def _lane_take(x, idxv):
    return lax.gather(x, idxv[:, None], _TAKE_DN, slice_sizes=(1,),
                      mode=lax.GatherScatterMode.PROMISE_IN_BOUNDS)


@functools.partial(
    pl.kernel,
    out_type=jax.ShapeDtypeStruct((N_PAIRS, W), f32),
    mesh=_MESH,
    scratch_types=[
        pltpu.VMEM((PCH, 128), i32),
        pltpu.VMEM((PCH, 128), i32),
        pltpu.VMEM((128, W), f32),
        pltpu.VMEM((128, W), f32),
        pltpu.VMEM((128, W), f32),
        pltpu.VMEM((128, W), f32),
        pltpu.VMEM((128, W), f32),
        pltpu.VMEM((16,), f32),
        pltpu.SemaphoreType.DMA,
    ],
    compiler_params=_SC_PARAMS,
)
def _s2(idx_hbm, x_hbm, p_hbm, k_hbm, out_hbm,
        i0_v, i1_v, x0_v, a0_v, x1_v, a1_v, out_v, kv, sem):
    cid = lax.axis_index("c")
    sid = lax.axis_index("s")
    wid = sid * NC + cid

    pltpu.sync_copy(k_hbm, kv)
    pltpu.sync_copy(idx_hbm.at[0, wid], i0_v)
    pltpu.sync_copy(idx_hbm.at[1, wid], i1_v)

    lanes = lax.iota(i32, 16)
    six = jnp.full((16,), 6, i32)
    sh8 = jnp.minimum(lanes + 8, 15)
    sh3 = jnp.minimum(lanes + 3, 15)

    for j in range(PCH):
        d0 = pltpu.async_copy(x_hbm.at[i0_v.at[j]], x0_v, sem)
        d1 = pltpu.async_copy(p_hbm.at[i0_v.at[j]], a0_v, sem)
        d2 = pltpu.async_copy(x_hbm.at[i1_v.at[j]], x1_v, sem)
        d3 = pltpu.async_copy(p_hbm.at[i1_v.at[j]], a1_v, sem)
        d0.wait(); d1.wait(); d2.wait(); d3.wait()

        kcst = kv[...]

        def _pair(p, carry):
            v0 = a0_v[p, :]
            s0 = v0 + _lane_take(v0, sh8)
            rec0 = 1.0 / jnp.maximum(_lane_take(s0, six), 1.0)
            tl = s0 * rec0 + _lane_take(x0_v[p, :], sh8)
            v1 = a1_v[p, :]
            s1 = v1 + _lane_take(v1, sh8)
            rec1 = 1.0 / jnp.maximum(_lane_take(s1, six), 1.0)
            tr = s1 * rec1 + _lane_take(x1_v[p, :], sh8)
            out_v[p, :] = tl + _lane_take(tr, sh3) + kcst
            return carry
        lax.fori_loop(0, 128, _pair, 0)

        base = (wid * PCH + j) * 128
        pltpu.sync_copy(out_v, out_hbm.at[pl.ds(base, 128)])


def kernel(node_embeddings, W1, b1, Wl, Wr, b_sage, Wc, bc, idx, edge_index):
    ne = node_embeddings.astype(f32)
    X, Xm, K = _dense(ne, W1, b1.reshape(1, HIDDEN),
                      Wl, Wr, Wc, b_sage.reshape(1, HIDDEN), bc.reshape(1, 3))

    pad = EPAD - N_EDGES
    padv = jnp.broadcast_to(jnp.array([[0], [NPAD - 1]], i32), (2, pad))
    ei = jnp.concatenate([edge_index.astype(i32), padv], axis=1
                         ).reshape(2, NW, ECH, 128)
    zeros8 = jnp.zeros((NPAD, 8), f32)
    p = _s1(ei, Xm, zeros8)

    idxT = idx.astype(i32).T.reshape(2, NW, PCH, 128)
    out = _s2(idxT, X, p, K.reshape(16))
    return out[:, :3]
